# baseline (speedup 1.0000x reference)
import functools

import jax
import jax.numpy as jnp
from jax import lax
from jax.experimental import pallas as pl
from jax.experimental.pallas import tpu as pltpu, tpu_sc as plsc

B = 16384
F = 26
D = 32
NC = 2
NS = 16
NW = NC * NS
L = 16

TABLE_ROWS = 100000 * F
SUPER = TABLE_ROWS * D // 128
ROWS = B * F
RPW = ROWS // NW
IRPW = RPW // 128
PATR = 13
C = 128
NCH = RPW // C
OC = C // 4

TW = 512
FULL = TABLE_ROWS - TABLE_ROWS % TW
NSLAB = FULL // TW
SPW = -(-NSLAB // NW)
TAIL = TABLE_ROWS - FULL


def _tbody(tt_hbm, tail_hbm, rm_hbm, slab0, slab1, tbuf, ttail, sem0, sem1):
    wid = lax.axis_index("s") * NC + lax.axis_index("c")
    lanes = lax.iota(jnp.int32, L)
    lanes_d4 = lax.shift_right_logical(lanes, 2)
    lanes_m4x32 = (lanes & 3) * D

    def shuffle_dyn(slab, dst, g):
        rvec = lanes_d4 + g * (L // 4)
        cols = lanes + g * L
        for c in range(D):
            v = plsc.load_gather(slab, [jnp.full((L,), c, jnp.int32), cols])
            plsc.store_scatter(dst, [rvec, lanes_m4x32 + c], v)

    def do_chunk(ch, slab, sem):
        def grp(g, carry):
            shuffle_dyn(slab, tbuf, g)
            return carry
        lax.fori_loop(0, TW // L, grp, 0)
        rm0 = pl.multiple_of(ch * (TW // 4), 8)
        pltpu.sync_copy(tbuf, rm_hbm.at[pl.ds(rm0, TW // 4)])

    def start(ch, slab, sem):
        c0 = pl.multiple_of(ch * TW, 128)
        return pltpu.async_copy(tt_hbm.at[:, pl.ds(c0, TW)], slab, sem)

    start(wid, slab0, sem0)

    def loop2(jj, carry):
        ch0 = (2 * jj) * NW + wid
        ch1 = (2 * jj + 1) * NW + wid

        @pl.when(ch1 < NSLAB)
        def _():
            start(ch1, slab1, sem1)

        @pl.when(ch0 < NSLAB)
        def _():
            pltpu.make_async_copy(tt_hbm.at[:, pl.ds(0, TW)], slab0,
                                  sem0).wait()
            do_chunk(ch0, slab0, sem0)

        ch2 = (2 * jj + 2) * NW + wid

        @pl.when(ch2 < NSLAB)
        def _():
            start(ch2, slab0, sem0)

        @pl.when(ch1 < NSLAB)
        def _():
            pltpu.make_async_copy(tt_hbm.at[:, pl.ds(0, TW)], slab1,
                                  sem1).wait()
            do_chunk(ch1, slab1, sem1)

        return carry

    lax.fori_loop(0, (SPW + 1) // 2, loop2, 0)

    @pl.when(wid == 0)
    def _():
        pltpu.sync_copy(tail_hbm, ttail)
        pltpu.sync_copy(ttail, rm_hbm.at[pl.ds(FULL // 4, TAIL // 4)])


def _gbody(idx_hbm, pat_hbm, table_hbm, out_hbm,
           idx_v, sidx_v, pat_v, buf, obuf, gsem):
    wid = lax.axis_index("s") * NC + lax.axis_index("c")
    ibase = pl.multiple_of(wid * IRPW, 8)
    obase = pl.multiple_of(wid * (RPW // 4), 8)

    pltpu.sync_copy(idx_hbm.at[pl.ds(ibase, IRPW)], idx_v)
    pltpu.sync_copy(pat_hbm, pat_v)

    def add_rows(g2, carry):
        for jj in range(PATR):
            i = g2 * PATR + jj
            for j in range(8):
                sl = pl.ds(j * L, L)
                r = idx_v[i, sl] + pat_v[jj, sl]
                idx_v[i, sl] = r
                sidx_v[i, sl] = lax.shift_right_logical(r, 2)
        return carry

    lax.fori_loop(0, IRPW // PATR, add_rows, 0)

    lanes = lax.iota(jnp.int32, L)
    dst_col0 = (lanes & 3) * D
    lane_div4 = lax.shift_right_logical(lanes, 2)

    def chunk(k, carry):
        pltpu.async_copy(table_hbm.at[sidx_v.at[k]], buf, gsem).wait()
        for g in range(C // L):
            e_vec = lanes + (g * L)
            r_vec = idx_v[k, pl.ds(g * L, L)]
            src = (r_vec & 3) * D
            dst_row = lane_div4 + (g * (L // 4))
            dst_col = dst_col0
            for _ in range(D):
                v = plsc.load_gather(buf, [e_vec, src])
                plsc.store_scatter(obuf, [dst_row, dst_col], v)
                src = src + 1
                dst_col = dst_col + 1
        pltpu.sync_copy(obuf, out_hbm.at[pl.ds(obase + k * OC, OC)])
        return carry

    lax.fori_loop(0, NCH, chunk, 0)


@jax.jit
def _run(idx2d, pat2d, ttable, tail16):
    mesh = plsc.VectorSubcoreMesh(core_axis_name="c", subcore_axis_name="s")
    params = pltpu.CompilerParams(
        needs_layout_passes=False, use_tc_tiling_on_sc=True)
    rm = pl.kernel(
        _tbody,
        out_type=jax.ShapeDtypeStruct((SUPER, 128), jnp.float32),
        mesh=mesh,
        scratch_types=[
            pltpu.VMEM((D, TW), jnp.float32),
            pltpu.VMEM((D, TW), jnp.float32),
            pltpu.VMEM((TW // 4, 128), jnp.float32),
            pltpu.VMEM((TAIL // 4, 128), jnp.float32),
            pltpu.SemaphoreType.DMA,
            pltpu.SemaphoreType.DMA,
        ],
        compiler_params=params,
    )(ttable, tail16)
    return pl.kernel(
        _gbody,
        out_type=jax.ShapeDtypeStruct((ROWS * D // 128, 128), jnp.float32),
        mesh=mesh,
        scratch_types=[
            pltpu.VMEM((IRPW, 128), jnp.int32),
            pltpu.VMEM((IRPW, 128), jnp.int32),
            pltpu.VMEM((PATR, 128), jnp.int32),
            pltpu.VMEM((C, 128), jnp.float32),
            pltpu.VMEM((OC, 128), jnp.float32),
            pltpu.SemaphoreType.DMA,
        ],
        compiler_params=params,
    )(idx2d, pat2d, rm)


def kernel(sparse_indices, offsets, embed_table):
    idx2d = sparse_indices.reshape(ROWS // 128, 128)
    pat2d = jnp.tile(offsets.reshape(F), PATR * 128 // F).reshape(PATR, 128)
    tail16 = embed_table[FULL:, :].reshape(TAIL // 4, 128)
    out = _run(idx2d, pat2d, embed_table.T, tail16)
    return out.reshape(B, F, D)


# speedup vs baseline: 1.6612x; 1.6612x over previous
import functools

import jax
import jax.numpy as jnp
from jax import lax
from jax.experimental import pallas as pl
from jax.experimental.pallas import tpu as pltpu, tpu_sc as plsc

B = 16384
F = 26
D = 32
NC = 2
NS = 16
NW = NC * NS
L = 16

ROWS = B * F
RPW = ROWS // NW
PAT = 208
GROUPS = RPW // PAT
C = 832
NCH = RPW // C


def _body(idx_hbm, pat_hbm, table_hbm, out_hbm,
          idx_v, pat_v, buf0, buf1, gsem, ssem):
    wid = lax.axis_index("s") * NC + lax.axis_index("c")
    base = wid * RPW

    pltpu.sync_copy(idx_hbm.at[pl.ds(base, RPW)], idx_v)
    pltpu.sync_copy(pat_hbm, pat_v)

    pat_regs = [pat_v[pl.ds(j * L, L)] for j in range(PAT // L)]

    def add_group(g, carry):
        s0 = g * PAT
        for j in range(PAT // L):
            sl = pl.ds(s0 + j * L, L)
            idx_v[sl] = idx_v[sl] + pat_regs[j]
        return carry

    lax.fori_loop(0, GROUPS, add_group, 0)

    bufs = [buf0, buf1]
    for k in range(NCH):
        buf = bufs[k % 2]
        pltpu.async_copy(
            table_hbm.at[idx_v.at[pl.ds(k * C, C)]], buf, gsem
        ).wait()
        pltpu.sync_copy(buf, out_hbm.at[pl.ds(base + k * C, C)])


@jax.jit
def _run(idx_flat, pat, table):
    mesh = plsc.VectorSubcoreMesh(core_axis_name="c", subcore_axis_name="s")
    return pl.kernel(
        _body,
        out_type=jax.ShapeDtypeStruct((ROWS, D), jnp.float32),
        mesh=mesh,
        scratch_types=[
            pltpu.VMEM((RPW,), jnp.int32),
            pltpu.VMEM((PAT,), jnp.int32),
            pltpu.VMEM((C, D), jnp.float32),
            pltpu.VMEM((C, D), jnp.float32),
            pltpu.SemaphoreType.DMA,
            pltpu.SemaphoreType.DMA,
        ],
        compiler_params=pltpu.CompilerParams(use_tc_tiling_on_sc=False),
    )(idx_flat, pat, table)


def kernel(sparse_indices, offsets, embed_table):
    idx_flat = sparse_indices.reshape(ROWS)
    pat = jnp.tile(offsets.reshape(F), L // 2)
    out = _run(idx_flat, pat, embed_table)
    return out.reshape(B, F, D)
